# baseline (speedup 1.0000x reference)
import functools

import jax
import jax.numpy as jnp
from jax import lax
from jax.experimental import pallas as pl
from jax.experimental.pallas import tpu as pltpu
from jax.experimental.pallas import tpu_sc as plsc

NUM_NODES = 1000000
STATE_DIM = 64
BATCH = 16384

_info = plsc.get_sparse_core_info()
_NC, _NS = _info.num_cores, _info.num_subcores
_NW = _NC * _NS
_B_PER_W = BATCH // _NW


def _make_gather():
    mesh = plsc.VectorSubcoreMesh(core_axis_name="c", subcore_axis_name="s")

    @functools.partial(
        pl.kernel,
        mesh=mesh,
        out_type=(
            jax.ShapeDtypeStruct((BATCH, STATE_DIM), jnp.float32),
            jax.ShapeDtypeStruct((BATCH,), jnp.int32),
        ),
        scratch_types=[
            pltpu.VMEM((_B_PER_W,), jnp.int32),
            pltpu.VMEM((_B_PER_W, STATE_DIM), jnp.float32),
            pltpu.VMEM((_B_PER_W,), jnp.int32),
            pltpu.SemaphoreType.DMA,
            pltpu.SemaphoreType.DMA,
        ],
        compiler_params=pltpu.CompilerParams(use_tc_tiling_on_sc=False),
    )
    def k(mem_hbm, lu_hbm, nid_hbm, mem_out, lu_out, idx_v, rows_v, lu_v,
          sem_rows, sem_lu):
        wid = lax.axis_index("s") * _NC + lax.axis_index("c")
        base = wid * _B_PER_W
        pltpu.sync_copy(nid_hbm.at[pl.ds(base, _B_PER_W)], idx_v)
        rows_cp = pltpu.async_copy(mem_hbm.at[idx_v], rows_v, sem_rows)
        lu_cp = pltpu.async_copy(lu_hbm.at[idx_v], lu_v, sem_lu)
        rows_cp.wait()
        pltpu.sync_copy(rows_v, mem_out.at[pl.ds(base, _B_PER_W)])
        lu_cp.wait()
        pltpu.sync_copy(lu_v, lu_out.at[pl.ds(base, _B_PER_W)])

    return k


_gather = _make_gather()


def kernel(memory, last_update, n_id):
    return _gather(memory, last_update, n_id.astype(jnp.int32))


# speedup vs baseline: 1.6899x; 1.6899x over previous
import functools

import jax
import jax.numpy as jnp
from jax import lax
from jax.experimental import pallas as pl
from jax.experimental.pallas import tpu as pltpu
from jax.experimental.pallas import tpu_sc as plsc

NUM_NODES = 1000000
STATE_DIM = 64
BATCH = 16384

_info = plsc.get_sparse_core_info()
_NC, _NS, _L = _info.num_cores, _info.num_subcores, _info.num_lanes
_NW = _NC * _NS
_B_PER_W = BATCH // _NW
_K = 16
_NG = _B_PER_W // _K


def _make_gather():
    mesh = plsc.VectorSubcoreMesh(core_axis_name="c", subcore_axis_name="s")

    @functools.partial(
        pl.kernel,
        mesh=mesh,
        out_type=(
            jax.ShapeDtypeStruct((BATCH, STATE_DIM), jnp.float32),
            jax.ShapeDtypeStruct((BATCH,), jnp.int32),
        ),
        scratch_types=[
            pltpu.VMEM((_B_PER_W,), jnp.int32),
            pltpu.VMEM((_B_PER_W, STATE_DIM), jnp.float32),
            pltpu.VMEM((_B_PER_W,), jnp.int32),
            pltpu.SemaphoreType.DMA,
            pltpu.SemaphoreType.DMA,
            pltpu.SemaphoreType.DMA,
        ],
    )
    def k(mem_hbm, lu_hbm, nid_hbm, mem_out, lu_out,
          idx_v, rows_v, lu_v, s_a, s_b, s_lu):
        wid = lax.axis_index("s") * _NC + lax.axis_index("c")
        base = wid * _B_PER_W
        pltpu.sync_copy(nid_hbm.at[pl.ds(base, _B_PER_W)], idx_v)
        lu_cp = pltpu.async_copy(lu_hbm.at[idx_v], lu_v, s_lu)

        def fire(g, sem):
            vec = idx_v[pl.ds(g * _K, _K)]
            for b in range(_K):
                v = vec[b]
                pltpu.async_copy(mem_hbm.at[v], rows_v.at[g * _K + b], sem)

        def drain(sem):
            pltpu.make_async_copy(
                mem_hbm.at[pl.ds(0, _K)], rows_v.at[pl.ds(0, _K)], sem
            ).wait()

        fire(0, s_a)

        def body(g2, _):
            g = g2 * 2
            fire(g + 1, s_b)
            drain(s_a)

            @pl.when(g + 2 < _NG)
            def _fire_next():
                fire(g + 2, s_a)

            drain(s_b)
            return _

        lax.fori_loop(0, _NG // 2, body, 0)

        pltpu.sync_copy(rows_v, mem_out.at[pl.ds(base, _B_PER_W)])
        lu_cp.wait()
        pltpu.sync_copy(lu_v, lu_out.at[pl.ds(base, _B_PER_W)])

    return k


_gather = _make_gather()


def kernel(memory, last_update, n_id):
    return _gather(memory, last_update, n_id.astype(jnp.int32))
